# baseline (speedup 1.0000x reference)
import jax
import jax.numpy as jnp
from jax.experimental import pallas as pl
from jax.experimental.pallas import tpu as pltpu

_BM = 1024
_N_TOKENS = 16384
_D_MODEL = 2048
_N_EXPERTS = 64


def _router_block(x_ref, w_ref, b_ref, o_ref):
    o_ref[...] = (
        jax.lax.dot_general(
            x_ref[...].astype(jnp.bfloat16),
            w_ref[...].astype(jnp.bfloat16),
            dimension_numbers=(((1,), (1,)), ((), ())),
            preferred_element_type=jnp.float32,
        )
        + b_ref[...]
    )


@jax.jit
def kernel(x, W, b):
    b2 = jax.lax.reshape(b, (1, _N_EXPERTS))
    return pl.pallas_call(
        _router_block,
        grid=(_N_TOKENS // _BM,),
        in_specs=[
            pl.BlockSpec((_BM, _D_MODEL), lambda i: (i, 0)),
            pl.BlockSpec((_N_EXPERTS, _D_MODEL), lambda i: (0, 0)),
            pl.BlockSpec((1, _N_EXPERTS), lambda i: (0, 0)),
        ],
        out_specs=pl.BlockSpec((_BM, _N_EXPERTS), lambda i: (i, 0)),
        out_shape=jax.ShapeDtypeStruct((_N_TOKENS, _N_EXPERTS), jnp.float32),
        compiler_params=pltpu.CompilerParams(
            dimension_semantics=("parallel",),
        ),
    )(x, W, b2)
